# Initial kernel scaffold; baseline (speedup 1.0000x reference)
import jax
import jax.numpy as jnp
from jax.experimental import pallas as pl


def kernel(proj_values, view_idx, point_idx, scenepoint_features, view_features, global_features, ln_sp_g, ln_sp_b, ln_v_g, ln_v_b, ln_g_g, ln_g_b, W_proj, b_proj, W_sp, W_view, W_glob):
    raise NotImplementedError("write your pallas kernel here")


# speedup vs baseline: 2.1615x; 2.1615x over previous
import functools

import jax
import jax.numpy as jnp
from jax import lax
from jax.experimental import pallas as pl
from jax.experimental.pallas import tpu as pltpu
from jax.experimental.pallas import tpu_sc as plsc

NNZ = 320000
NP = 10000
NV = 200
D = 128

_NC = 2
_NS = 16
_NW = _NC * _NS
_EPW = NNZ // _NW
_C = 80
_NCHUNK = _EPW // _C


def _ln_relu(x, g, b):
    m = jnp.mean(x, axis=-1, keepdims=True)
    v = jnp.mean((x - m) ** 2, axis=-1, keepdims=True)
    y = (x - m) * jax.lax.rsqrt(v + 1e-5) * g + b
    return jnp.maximum(y, 0.0)


def _prep_body(sp_ref, vw_ref, gl_ref, g_sp, b_sp, g_v, b_v, g_g, b_g,
               w_sp, w_view, w_glob, b_proj, sp4_ref, vw4_ref, bias4_ref):
    sp = _ln_relu(sp_ref[...], g_sp[...], b_sp[...])
    vw = _ln_relu(vw_ref[...], g_v[...], b_v[...])
    gl = _ln_relu(gl_ref[...], g_g[...], b_g[...])
    dn = (((1,), (1,)), ((), ()))
    sp4_ref[...] = 0.25 * lax.dot_general(sp, w_sp[...], dn,
                                          preferred_element_type=jnp.float32)
    vw4_ref[...] = 0.25 * lax.dot_general(vw, w_view[...], dn,
                                          preferred_element_type=jnp.float32)
    glp = lax.dot_general(gl, w_glob[...], dn,
                          preferred_element_type=jnp.float32)
    bias4_ref[...] = 0.25 * (glp + b_proj[...])


def _sc_gather(sp4_hbm, vw4_hbm, pi_hbm, vi_hbm, gsp_hbm, gvw_hbm,
               pi_v, vi_v, spbuf, vwbuf, sem1, sem2):
    wid = lax.axis_index("s") * _NC + lax.axis_index("c")
    base = wid * _EPW
    pltpu.sync_copy(pi_hbm.at[pl.ds(base, _EPW)], pi_v)
    pltpu.sync_copy(vi_hbm.at[pl.ds(base, _EPW)], vi_v)

    def body(k, carry):
        off = k * _C
        cp1 = pltpu.async_copy(sp4_hbm.at[pi_v.at[pl.ds(off, _C)]], spbuf, sem1)
        cp2 = pltpu.async_copy(vw4_hbm.at[vi_v.at[pl.ds(off, _C)]], vwbuf, sem2)
        cp1.wait()
        cp2.wait()
        pltpu.sync_copy(spbuf, gsp_hbm.at[pl.ds(base + off, _C)])
        pltpu.sync_copy(vwbuf, gvw_hbm.at[pl.ds(base + off, _C)])
        return carry

    lax.fori_loop(0, _NCHUNK, body, 0)


def _main_body(x_ref, gsp_ref, gvw_ref, w_ref, bias_ref, o_ref):
    dn = (((1,), (1,)), ((), ()))
    acc = lax.dot_general(x_ref[...], w_ref[...], dn,
                          preferred_element_type=jnp.float32)
    o_ref[...] = acc * 0.25 + (gsp_ref[...] + gvw_ref[...] + bias_ref[...])


def kernel(proj_values, view_idx, point_idx, scenepoint_features,
           view_features, global_features, ln_sp_g, ln_sp_b, ln_v_g, ln_v_b,
           ln_g_g, ln_g_b, W_proj, b_proj, W_sp, W_view, W_glob):
    row = lambda x: x.reshape(1, D)
    sp4, vw4, bias4 = pl.pallas_call(
        _prep_body,
        out_shape=[
            jax.ShapeDtypeStruct((NP, D), jnp.float32),
            jax.ShapeDtypeStruct((NV, D), jnp.float32),
            jax.ShapeDtypeStruct((1, D), jnp.float32),
        ],
    )(scenepoint_features, view_features, row(global_features),
      row(ln_sp_g), row(ln_sp_b), row(ln_v_g), row(ln_v_b),
      row(ln_g_g), row(ln_g_b), W_sp, W_view, W_glob, row(b_proj))

    mesh = plsc.VectorSubcoreMesh(core_axis_name="c", subcore_axis_name="s")
    gsp, gvw = pl.kernel(
        _sc_gather,
        mesh=mesh,
        out_type=[
            jax.ShapeDtypeStruct((NNZ, D), jnp.float32),
            jax.ShapeDtypeStruct((NNZ, D), jnp.float32),
        ],
        scratch_types=[
            pltpu.VMEM((_EPW,), jnp.int32),
            pltpu.VMEM((_EPW,), jnp.int32),
            pltpu.VMEM((_C, D), jnp.float32),
            pltpu.VMEM((_C, D), jnp.float32),
            pltpu.SemaphoreType.DMA,
            pltpu.SemaphoreType.DMA,
        ],
    )(sp4, vw4, point_idx.astype(jnp.int32), view_idx.astype(jnp.int32))

    R = 1600
    out = pl.pallas_call(
        _main_body,
        grid=(NNZ // R,),
        in_specs=[
            pl.BlockSpec((R, D), lambda i: (i, 0)),
            pl.BlockSpec((R, D), lambda i: (i, 0)),
            pl.BlockSpec((R, D), lambda i: (i, 0)),
            pl.BlockSpec((D, D), lambda i: (0, 0)),
            pl.BlockSpec((1, D), lambda i: (0, 0)),
        ],
        out_specs=pl.BlockSpec((R, D), lambda i: (i, 0)),
        out_shape=jax.ShapeDtypeStruct((NNZ, D), jnp.float32),
    )(proj_values, gsp, gvw, W_proj, bias4)
    return out


# speedup vs baseline: 3.9256x; 1.8162x over previous
import jax
import jax.numpy as jnp
from jax import lax
from jax.experimental import pallas as pl
from jax.experimental.pallas import tpu as pltpu
from jax.experimental.pallas import tpu_sc as plsc

NNZ = 320000
NP = 10000
NV = 200
NVP = 256
D = 128

_NC = 2
_NS = 16
_NW = _NC * _NS
_EPW = NNZ // _NW
_C = 80
_NCHUNK = _EPW // _C

_R = 1600


def _ln_relu(x, g, b):
    m = jnp.mean(x, axis=-1, keepdims=True)
    v = jnp.mean((x - m) ** 2, axis=-1, keepdims=True)
    y = (x - m) * jax.lax.rsqrt(v + 1e-5) * g + b
    return jnp.maximum(y, 0.0)


def _prep_body(sp_ref, vw_ref, gl_ref, g_sp, b_sp, g_v, b_v, g_g, b_g,
               w_sp, w_view, w_glob, b_proj, sp4_ref, vw4_ref, bias4_ref):
    sp = _ln_relu(sp_ref[...], g_sp[...], b_sp[...])
    vw = _ln_relu(vw_ref[...], g_v[...], b_v[...])
    gl = _ln_relu(gl_ref[...], g_g[...], b_g[...])
    dn = (((1,), (1,)), ((), ()))
    sp4_ref[...] = 0.25 * lax.dot_general(sp, w_sp[...], dn,
                                          preferred_element_type=jnp.float32)
    vw4_ref[...] = 0.25 * lax.dot_general(vw, w_view[...], dn,
                                          preferred_element_type=jnp.float32)
    glp = lax.dot_general(gl, w_glob[...], dn,
                          preferred_element_type=jnp.float32)
    bias4_ref[...] = 0.25 * (glp + b_proj[...])


def _sc_gather(sp4_hbm, pi_hbm, gsp_hbm, pi_v, buf0, buf1, sem0, sem1):
    wid = lax.axis_index("s") * _NC + lax.axis_index("c")
    base = wid * _EPW
    pltpu.sync_copy(pi_hbm.at[pl.ds(base, _EPW)], pi_v)

    def start(c, buf, sem):
        pltpu.async_copy(sp4_hbm.at[pi_v.at[pl.ds(c * _C, _C)]], buf, sem)

    def wait(buf, sem):
        pltpu.make_async_copy(sp4_hbm.at[pl.ds(0, _C)], buf, sem).wait()

    def scatter(c, buf):
        pltpu.sync_copy(buf, gsp_hbm.at[pl.ds(base + c * _C, _C)])

    start(0, buf0, sem0)

    def body(k, carry):
        c0 = 2 * k
        start(c0 + 1, buf1, sem1)
        wait(buf0, sem0)
        scatter(c0, buf0)
        start(c0 + 2, buf0, sem0)
        wait(buf1, sem1)
        scatter(c0 + 1, buf1)
        return carry

    lax.fori_loop(0, (_NCHUNK - 1) // 2, body, 0)
    wait(buf0, sem0)
    scatter(_NCHUNK - 1, buf0)


def _main_body(x_ref, gsp_ref, vi_ref, w_ref, vwt_ref, bias_ref, o_ref):
    dn = (((1,), (1,)), ((), ()))
    acc = lax.dot_general(x_ref[...], w_ref[...], dn,
                          preferred_element_type=jnp.float32)
    vi = vi_ref[0, 0, :]
    col = lax.broadcasted_iota(jnp.int32, (_R, NVP), 1)
    oh = (col == vi.reshape(_R, 1)).astype(jnp.bfloat16)
    vwterm = lax.dot_general(oh, vwt_ref[...], (((1,), (0,)), ((), ())),
                             preferred_element_type=jnp.float32)
    o_ref[...] = acc * 0.25 + (vwterm + (gsp_ref[...] + bias_ref[...]))


def kernel(proj_values, view_idx, point_idx, scenepoint_features,
           view_features, global_features, ln_sp_g, ln_sp_b, ln_v_g, ln_v_b,
           ln_g_g, ln_g_b, W_proj, b_proj, W_sp, W_view, W_glob):
    row = lambda x: x.reshape(1, D)
    sp4, vw4, bias4 = pl.pallas_call(
        _prep_body,
        out_shape=[
            jax.ShapeDtypeStruct((NP, D), jnp.float32),
            jax.ShapeDtypeStruct((NV, D), jnp.float32),
            jax.ShapeDtypeStruct((1, D), jnp.float32),
        ],
    )(scenepoint_features, view_features, row(global_features),
      row(ln_sp_g), row(ln_sp_b), row(ln_v_g), row(ln_v_b),
      row(ln_g_g), row(ln_g_b), W_sp, W_view, W_glob, row(b_proj))

    mesh = plsc.VectorSubcoreMesh(core_axis_name="c", subcore_axis_name="s")
    gsp = pl.kernel(
        _sc_gather,
        mesh=mesh,
        out_type=jax.ShapeDtypeStruct((NNZ, D), jnp.float32),
        scratch_types=[
            pltpu.VMEM((_EPW,), jnp.int32),
            pltpu.VMEM((_C, D), jnp.float32),
            pltpu.VMEM((_C, D), jnp.float32),
            pltpu.SemaphoreType.DMA,
            pltpu.SemaphoreType.DMA,
        ],
    )(sp4, point_idx.astype(jnp.int32))

    vwt = jnp.pad(vw4, ((0, NVP - NV), (0, 0))).astype(jnp.bfloat16)
    vi3 = view_idx.astype(jnp.int32).reshape(NNZ // _R, 1, _R)
    out = pl.pallas_call(
        _main_body,
        grid=(NNZ // _R,),
        in_specs=[
            pl.BlockSpec((_R, D), lambda i: (i, 0)),
            pl.BlockSpec((_R, D), lambda i: (i, 0)),
            pl.BlockSpec((1, 1, _R), lambda i: (i, 0, 0)),
            pl.BlockSpec((D, D), lambda i: (0, 0)),
            pl.BlockSpec((NVP, D), lambda i: (0, 0)),
            pl.BlockSpec((1, D), lambda i: (0, 0)),
        ],
        out_specs=pl.BlockSpec((_R, D), lambda i: (i, 0)),
        out_shape=jax.ShapeDtypeStruct((NNZ, D), jnp.float32),
    )(proj_values, gsp, vi3, W_proj, vwt, bias4)
    return out


# speedup vs baseline: 4.3626x; 1.1113x over previous
import functools

import jax
import jax.numpy as jnp
from jax import lax
from jax.experimental import pallas as pl
from jax.experimental.pallas import tpu as pltpu
from jax.experimental.pallas import tpu_sc as plsc

NNZ = 320000
NP = 10000
NV = 200
NVP = 256
D = 128

_S = 5
_SLICE = NNZ // _S

_NC = 2
_NS = 16
_NW = _NC * _NS
_EPW = _SLICE // _NW
_C = 80
_NCHUNK = _EPW // _C

_R = 1600
_BPS = _SLICE // _R


def _ln_relu(x, g, b):
    m = jnp.mean(x, axis=-1, keepdims=True)
    v = jnp.mean((x - m) ** 2, axis=-1, keepdims=True)
    y = (x - m) * jax.lax.rsqrt(v + 1e-5) * g + b
    return jnp.maximum(y, 0.0)


def _prep_body(sp_ref, vw_ref, gl_ref, g_sp, b_sp, g_v, b_v, g_g, b_g,
               w_sp, w_view, w_glob, b_proj, sp4_ref, vw4_ref, bias4_ref):
    sp = _ln_relu(sp_ref[...], g_sp[...], b_sp[...])
    vw = _ln_relu(vw_ref[...], g_v[...], b_v[...])
    gl = _ln_relu(gl_ref[...], g_g[...], b_g[...])
    dn = (((1,), (1,)), ((), ()))
    sp4_ref[...] = 0.25 * lax.dot_general(sp, w_sp[...], dn,
                                          preferred_element_type=jnp.float32)
    vw4_ref[...] = 0.25 * lax.dot_general(vw, w_view[...], dn,
                                          preferred_element_type=jnp.float32)
    glp = lax.dot_general(gl, w_glob[...], dn,
                          preferred_element_type=jnp.float32)
    bias4_ref[...] = 0.25 * (glp + b_proj[...])


def _sc_gather(s, sp4_hbm, pi_hbm, gsp_hbm, pi_v, buf0, buf1, sem0, sem1):
    wid = lax.axis_index("s") * _NC + lax.axis_index("c")
    obase = wid * _EPW
    ibase = s * _SLICE + obase
    pltpu.sync_copy(pi_hbm.at[pl.ds(ibase, _EPW)], pi_v)

    def start(c, buf, sem):
        pltpu.async_copy(sp4_hbm.at[pi_v.at[pl.ds(c * _C, _C)]], buf, sem)

    def wait(buf, sem):
        pltpu.make_async_copy(sp4_hbm.at[pl.ds(0, _C)], buf, sem).wait()

    def scatter(c, buf):
        pltpu.sync_copy(buf, gsp_hbm.at[pl.ds(obase + c * _C, _C)])

    start(0, buf0, sem0)

    def body(k, carry):
        c0 = 2 * k
        start(c0 + 1, buf1, sem1)
        wait(buf0, sem0)
        scatter(c0, buf0)
        start(c0 + 2, buf0, sem0)
        wait(buf1, sem1)
        scatter(c0 + 1, buf1)
        return carry

    lax.fori_loop(0, (_NCHUNK - 1) // 2, body, 0)
    wait(buf0, sem0)
    scatter(_NCHUNK - 1, buf0)


def _main_body(x_ref, gsp_ref, vi_ref, w_ref, vwt_ref, bias_ref, *rest):
    o_ref = rest[-1]
    dn = (((1,), (1,)), ((), ()))
    acc = lax.dot_general(x_ref[...], w_ref[...], dn,
                          preferred_element_type=jnp.float32)
    vi = vi_ref[0, 0, :]
    col = lax.broadcasted_iota(jnp.int32, (_R, NVP), 1)
    oh = (col == vi.reshape(_R, 1)).astype(jnp.bfloat16)
    vwterm = lax.dot_general(oh, vwt_ref[...], (((1,), (0,)), ((), ())),
                             preferred_element_type=jnp.float32)
    o_ref[...] = acc * 0.25 + (vwterm + (gsp_ref[...] + bias_ref[...]))


def kernel(proj_values, view_idx, point_idx, scenepoint_features,
           view_features, global_features, ln_sp_g, ln_sp_b, ln_v_g, ln_v_b,
           ln_g_g, ln_g_b, W_proj, b_proj, W_sp, W_view, W_glob):
    row = lambda x: x.reshape(1, D)
    sp4, vw4, bias4 = pl.pallas_call(
        _prep_body,
        out_shape=[
            jax.ShapeDtypeStruct((NP, D), jnp.float32),
            jax.ShapeDtypeStruct((NV, D), jnp.float32),
            jax.ShapeDtypeStruct((1, D), jnp.float32),
        ],
    )(scenepoint_features, view_features, row(global_features),
      row(ln_sp_g), row(ln_sp_b), row(ln_v_g), row(ln_v_b),
      row(ln_g_g), row(ln_g_b), W_sp, W_view, W_glob, row(b_proj))

    mesh = plsc.VectorSubcoreMesh(core_axis_name="c", subcore_axis_name="s")
    pi32 = point_idx.astype(jnp.int32)
    gsp_slices = []
    for s in range(_S):
        gsp_slices.append(pl.kernel(
            functools.partial(_sc_gather, s),
            mesh=mesh,
            out_type=jax.ShapeDtypeStruct((_SLICE, D), jnp.float32),
            scratch_types=[
                pltpu.VMEM((_EPW,), jnp.int32),
                pltpu.VMEM((_C, D), jnp.float32),
                pltpu.VMEM((_C, D), jnp.float32),
                pltpu.SemaphoreType.DMA,
                pltpu.SemaphoreType.DMA,
            ],
        )(sp4, pi32))

    vwt = jnp.pad(vw4, ((0, NVP - NV), (0, 0))).astype(jnp.bfloat16)
    vi3 = view_idx.astype(jnp.int32).reshape(NNZ // _R, 1, _R)

    out = None
    for s in range(_S):
        base_specs = [
            pl.BlockSpec((_R, D), lambda i, s=s: (i + s * _BPS, 0)),
            pl.BlockSpec((_R, D), lambda i: (i, 0)),
            pl.BlockSpec((1, 1, _R), lambda i, s=s: (i + s * _BPS, 0, 0)),
            pl.BlockSpec((D, D), lambda i: (0, 0)),
            pl.BlockSpec((NVP, D), lambda i: (0, 0)),
            pl.BlockSpec((1, D), lambda i: (0, 0)),
        ]
        if s == 0:
            main_call = pl.pallas_call(
                _main_body,
                grid=(_BPS,),
                in_specs=base_specs,
                out_specs=pl.BlockSpec((_R, D), lambda i: (i, 0)),
                out_shape=jax.ShapeDtypeStruct((NNZ, D), jnp.float32),
            )
            out = main_call(proj_values, gsp_slices[s], vi3, W_proj, vwt,
                            bias4)
        else:
            main_call = pl.pallas_call(
                _main_body,
                grid=(_BPS,),
                in_specs=base_specs + [pl.BlockSpec(memory_space=pl.ANY)],
                out_specs=pl.BlockSpec(
                    (_R, D), lambda i, s=s: (i + s * _BPS, 0)),
                out_shape=jax.ShapeDtypeStruct((NNZ, D), jnp.float32),
                input_output_aliases={6: 0},
            )
            out = main_call(proj_values, gsp_slices[s], vi3, W_proj, vwt,
                            bias4, out)
    return out


# speedup vs baseline: 4.8037x; 1.1011x over previous
import functools

import jax
import jax.numpy as jnp
from jax import lax
from jax.experimental import pallas as pl
from jax.experimental.pallas import tpu as pltpu
from jax.experimental.pallas import tpu_sc as plsc

NNZ = 320000
NP = 10000
NV = 200
NVP = 256
D = 128

_S = 5
_SLICE = NNZ // _S

_NC = 2
_NS = 16
_NW = _NC * _NS
_EPW = _SLICE // _NW
_C = 80
_NCHUNK = _EPW // _C

_R = 1600
_BPS = _SLICE // _R


def _ln_relu(x, g, b):
    m = jnp.mean(x, axis=-1, keepdims=True)
    v = jnp.mean((x - m) ** 2, axis=-1, keepdims=True)
    y = (x - m) * jax.lax.rsqrt(v + 1e-5) * g + b
    return jnp.maximum(y, 0.0)


def _prep_body(sp_ref, vw_ref, gl_ref, g_sp, b_sp, g_v, b_v, g_g, b_g,
               w_sp, w_view, w_glob, b_proj, sp4_ref, vw4_ref, bias4_ref):
    sp = _ln_relu(sp_ref[...], g_sp[...], b_sp[...])
    vw = _ln_relu(vw_ref[...], g_v[...], b_v[...])
    gl = _ln_relu(gl_ref[...], g_g[...], b_g[...])
    dn = (((1,), (1,)), ((), ()))
    sp4_ref[...] = 0.25 * lax.dot_general(sp, w_sp[...], dn,
                                          preferred_element_type=jnp.float32)
    vw4_ref[...] = 0.25 * lax.dot_general(vw, w_view[...], dn,
                                          preferred_element_type=jnp.float32)
    glp = lax.dot_general(gl, w_glob[...], dn,
                          preferred_element_type=jnp.float32)
    bias4_ref[...] = 0.25 * (glp + b_proj[...])


def _sc_gather(s, sp4_hbm, pi_hbm, gsp_hbm, tab, pi_v, buf0, buf1, sem0,
               sem1):
    sid = lax.axis_index("s")
    wid = sid * _NC + lax.axis_index("c")
    obase = wid * _EPW
    ibase = s * _SLICE + obase
    pltpu.sync_copy(pi_hbm.at[pl.ds(ibase, _EPW)], pi_v)

    @pl.when(sid == 0)
    def _load_table():
        pltpu.sync_copy(sp4_hbm, tab)

    plsc.subcore_barrier()

    def start(c, buf, sem):
        pltpu.async_copy(tab.at[pi_v.at[pl.ds(c * _C, _C)]], buf, sem)

    def wait(buf, sem):
        pltpu.make_async_copy(sp4_hbm.at[pl.ds(0, _C)], buf, sem).wait()

    def scatter(c, buf):
        pltpu.sync_copy(buf, gsp_hbm.at[pl.ds(obase + c * _C, _C)])

    start(0, buf0, sem0)

    def body(k, carry):
        c0 = 2 * k
        start(c0 + 1, buf1, sem1)
        wait(buf0, sem0)
        scatter(c0, buf0)
        start(c0 + 2, buf0, sem0)
        wait(buf1, sem1)
        scatter(c0 + 1, buf1)
        return carry

    lax.fori_loop(0, (_NCHUNK - 1) // 2, body, 0)
    wait(buf0, sem0)
    scatter(_NCHUNK - 1, buf0)


def _main_body(x_ref, gsp_ref, vi_ref, w_ref, vwt_ref, bias_ref, *rest):
    o_ref = rest[-1]
    dn = (((1,), (1,)), ((), ()))
    acc = lax.dot_general(x_ref[...], w_ref[...], dn,
                          preferred_element_type=jnp.float32)
    vi = vi_ref[0, 0, :]
    col = lax.broadcasted_iota(jnp.int32, (_R, NVP), 1)
    oh = (col == vi.reshape(_R, 1)).astype(jnp.bfloat16)
    vwterm = lax.dot_general(oh, vwt_ref[...], (((1,), (0,)), ((), ())),
                             preferred_element_type=jnp.float32)
    o_ref[...] = acc * 0.25 + (vwterm + (gsp_ref[...] + bias_ref[...]))


def kernel(proj_values, view_idx, point_idx, scenepoint_features,
           view_features, global_features, ln_sp_g, ln_sp_b, ln_v_g, ln_v_b,
           ln_g_g, ln_g_b, W_proj, b_proj, W_sp, W_view, W_glob):
    row = lambda x: x.reshape(1, D)
    sp4, vw4, bias4 = pl.pallas_call(
        _prep_body,
        out_shape=[
            jax.ShapeDtypeStruct((NP, D), jnp.float32),
            jax.ShapeDtypeStruct((NV, D), jnp.float32),
            jax.ShapeDtypeStruct((1, D), jnp.float32),
        ],
    )(scenepoint_features, view_features, row(global_features),
      row(ln_sp_g), row(ln_sp_b), row(ln_v_g), row(ln_v_b),
      row(ln_g_g), row(ln_g_b), W_sp, W_view, W_glob, row(b_proj))

    mesh = plsc.VectorSubcoreMesh(core_axis_name="c", subcore_axis_name="s")
    pi32 = point_idx.astype(jnp.int32)
    gsp_slices = []
    for s in range(_S):
        gsp_slices.append(pl.kernel(
            functools.partial(_sc_gather, s),
            mesh=mesh,
            out_type=jax.ShapeDtypeStruct((_SLICE, D), jnp.float32),
            scratch_types=[
                pltpu.VMEM_SHARED((NP, D), jnp.float32),
                pltpu.VMEM((_EPW,), jnp.int32),
                pltpu.VMEM((_C, D), jnp.float32),
                pltpu.VMEM((_C, D), jnp.float32),
                pltpu.SemaphoreType.DMA,
                pltpu.SemaphoreType.DMA,
            ],
        )(sp4, pi32))

    vwt = jnp.pad(vw4, ((0, NVP - NV), (0, 0))).astype(jnp.bfloat16)
    vi3 = view_idx.astype(jnp.int32).reshape(NNZ // _R, 1, _R)

    out = None
    for s in range(_S):
        base_specs = [
            pl.BlockSpec((_R, D), lambda i, s=s: (i + s * _BPS, 0)),
            pl.BlockSpec((_R, D), lambda i: (i, 0)),
            pl.BlockSpec((1, 1, _R), lambda i, s=s: (i + s * _BPS, 0, 0)),
            pl.BlockSpec((D, D), lambda i: (0, 0)),
            pl.BlockSpec((NVP, D), lambda i: (0, 0)),
            pl.BlockSpec((1, D), lambda i: (0, 0)),
        ]
        if s == 0:
            main_call = pl.pallas_call(
                _main_body,
                grid=(_BPS,),
                in_specs=base_specs,
                out_specs=pl.BlockSpec((_R, D), lambda i: (i, 0)),
                out_shape=jax.ShapeDtypeStruct((NNZ, D), jnp.float32),
            )
            out = main_call(proj_values, gsp_slices[s], vi3, W_proj, vwt,
                            bias4)
        else:
            main_call = pl.pallas_call(
                _main_body,
                grid=(_BPS,),
                in_specs=base_specs + [pl.BlockSpec(memory_space=pl.ANY)],
                out_specs=pl.BlockSpec(
                    (_R, D), lambda i, s=s: (i + s * _BPS, 0)),
                out_shape=jax.ShapeDtypeStruct((NNZ, D), jnp.float32),
                input_output_aliases={6: 0},
            )
            out = main_call(proj_values, gsp_slices[s], vi3, W_proj, vwt,
                            bias4, out)
    return out


# speedup vs baseline: 5.7533x; 1.1977x over previous
import functools

import jax
import jax.numpy as jnp
from jax import lax
from jax.experimental import pallas as pl
from jax.experimental.pallas import tpu as pltpu
from jax.experimental.pallas import tpu_sc as plsc

NNZ = 320000
NP = 10000
NV = 200
NVP = 256
D = 128

_S = 5
_SLICE = NNZ // _S

_NC = 2
_NS = 16
_NW = _NC * _NS
_EPW = _SLICE // _NW
_C = 80
_NCHUNK = _EPW // _C

_R = 3200
_BPS = _SLICE // _R


def _ln_relu(x, g, b):
    m = jnp.mean(x, axis=-1, keepdims=True)
    v = jnp.mean((x - m) ** 2, axis=-1, keepdims=True)
    y = (x - m) * jax.lax.rsqrt(v + 1e-5) * g + b
    return jnp.maximum(y, 0.0)


def _prep_body(sp_ref, vw_ref, gl_ref, g_sp, b_sp, g_v, b_v, g_g, b_g,
               w_sp, w_view, w_glob, b_proj, sp4_ref, vw4_ref, bias4_ref):
    sp = _ln_relu(sp_ref[...], g_sp[...], b_sp[...])
    vw = _ln_relu(vw_ref[...], g_v[...], b_v[...])
    gl = _ln_relu(gl_ref[...], g_g[...], b_g[...])
    dn = (((1,), (1,)), ((), ()))
    sp4_ref[...] = 0.25 * lax.dot_general(sp, w_sp[...], dn,
                                          preferred_element_type=jnp.float32)
    vw4_ref[...] = 0.25 * lax.dot_general(vw, w_view[...], dn,
                                          preferred_element_type=jnp.float32)
    glp = lax.dot_general(gl, w_glob[...], dn,
                          preferred_element_type=jnp.float32)
    bias4_ref[...] = 0.25 * (glp + b_proj[...])


def _sc_gather(s, sp4_hbm, pi_hbm, gsp_hbm, tab, pi_v, buf0, buf1, sem0,
               sem1):
    sid = lax.axis_index("s")
    wid = sid * _NC + lax.axis_index("c")
    obase = wid * _EPW
    ibase = s * _SLICE + obase
    pltpu.sync_copy(pi_hbm.at[pl.ds(ibase, _EPW)], pi_v)

    @pl.when(sid == 0)
    def _load_table():
        pltpu.sync_copy(sp4_hbm, tab)

    plsc.subcore_barrier()

    def start(c, buf, sem):
        pltpu.async_copy(tab.at[pi_v.at[pl.ds(c * _C, _C)]], buf, sem)

    def wait(buf, sem):
        pltpu.make_async_copy(sp4_hbm.at[pl.ds(0, _C)], buf, sem).wait()

    def scatter(c, buf):
        pltpu.sync_copy(buf, gsp_hbm.at[pl.ds(obase + c * _C, _C)])

    start(0, buf0, sem0)

    def body(k, carry):
        c0 = 2 * k
        start(c0 + 1, buf1, sem1)
        wait(buf0, sem0)
        scatter(c0, buf0)
        start(c0 + 2, buf0, sem0)
        wait(buf1, sem1)
        scatter(c0 + 1, buf1)
        return carry

    lax.fori_loop(0, (_NCHUNK - 1) // 2, body, 0)
    wait(buf0, sem0)
    scatter(_NCHUNK - 1, buf0)


def _main_body(x_ref, gsp_ref, vi_ref, w_ref, vwt_ref, bias_ref, *rest):
    o_ref = rest[-1]
    dn = (((1,), (1,)), ((), ()))
    acc = lax.dot_general(x_ref[...], w_ref[...], dn,
                          preferred_element_type=jnp.float32)
    vi = vi_ref[0, 0, :]
    col = lax.broadcasted_iota(jnp.int32, (_R, NVP), 1)
    oh = (col == vi.reshape(_R, 1)).astype(jnp.bfloat16)
    vwterm = lax.dot_general(oh, vwt_ref[...], (((1,), (0,)), ((), ())),
                             preferred_element_type=jnp.float32)
    o_ref[...] = acc * 0.25 + (vwterm + (gsp_ref[...] + bias_ref[...]))


def kernel(proj_values, view_idx, point_idx, scenepoint_features,
           view_features, global_features, ln_sp_g, ln_sp_b, ln_v_g, ln_v_b,
           ln_g_g, ln_g_b, W_proj, b_proj, W_sp, W_view, W_glob):
    row = lambda x: x.reshape(1, D)
    sp4, vw4, bias4 = pl.pallas_call(
        _prep_body,
        out_shape=[
            jax.ShapeDtypeStruct((NP, D), jnp.float32),
            jax.ShapeDtypeStruct((NV, D), jnp.float32),
            jax.ShapeDtypeStruct((1, D), jnp.float32),
        ],
    )(scenepoint_features, view_features, row(global_features),
      row(ln_sp_g), row(ln_sp_b), row(ln_v_g), row(ln_v_b),
      row(ln_g_g), row(ln_g_b), W_sp, W_view, W_glob, row(b_proj))

    mesh = plsc.VectorSubcoreMesh(core_axis_name="c", subcore_axis_name="s")
    pi32 = point_idx.astype(jnp.int32)
    gsp_slices = []
    for s in range(_S):
        gsp_slices.append(pl.kernel(
            functools.partial(_sc_gather, s),
            mesh=mesh,
            out_type=jax.ShapeDtypeStruct((_SLICE, D), jnp.float32),
            scratch_types=[
                pltpu.VMEM_SHARED((NP, D), jnp.float32),
                pltpu.VMEM((_EPW,), jnp.int32),
                pltpu.VMEM((_C, D), jnp.float32),
                pltpu.VMEM((_C, D), jnp.float32),
                pltpu.SemaphoreType.DMA,
                pltpu.SemaphoreType.DMA,
            ],
        )(sp4, pi32))

    vwt = jnp.pad(vw4, ((0, NVP - NV), (0, 0))).astype(jnp.bfloat16)
    vi3 = view_idx.astype(jnp.int32).reshape(NNZ // _R, 1, _R)

    out = None
    for s in range(_S):
        base_specs = [
            pl.BlockSpec((_R, D), lambda i, s=s: (i + s * _BPS, 0)),
            pl.BlockSpec((_R, D), lambda i: (i, 0)),
            pl.BlockSpec((1, 1, _R), lambda i, s=s: (i + s * _BPS, 0, 0)),
            pl.BlockSpec((D, D), lambda i: (0, 0)),
            pl.BlockSpec((NVP, D), lambda i: (0, 0)),
            pl.BlockSpec((1, D), lambda i: (0, 0)),
        ]
        if s == 0:
            main_call = pl.pallas_call(
                _main_body,
                grid=(_BPS,),
                in_specs=base_specs,
                out_specs=pl.BlockSpec((_R, D), lambda i: (i, 0)),
                out_shape=jax.ShapeDtypeStruct((NNZ, D), jnp.float32),
            )
            out = main_call(proj_values, gsp_slices[s], vi3, W_proj, vwt,
                            bias4)
        else:
            main_call = pl.pallas_call(
                _main_body,
                grid=(_BPS,),
                in_specs=base_specs + [pl.BlockSpec(memory_space=pl.ANY)],
                out_specs=pl.BlockSpec(
                    (_R, D), lambda i, s=s: (i + s * _BPS, 0)),
                out_shape=jax.ShapeDtypeStruct((NNZ, D), jnp.float32),
                input_output_aliases={6: 0},
            )
            out = main_call(proj_values, gsp_slices[s], vi3, W_proj, vwt,
                            bias4, out)
    return out


# speedup vs baseline: 6.1185x; 1.0635x over previous
import functools

import jax
import jax.numpy as jnp
from jax import lax
from jax.experimental import pallas as pl
from jax.experimental.pallas import tpu as pltpu
from jax.experimental.pallas import tpu_sc as plsc

NNZ = 320000
NP = 10000
NV = 200
NVP = 256
D = 128

_S = 5
_SLICE = NNZ // _S

_NC = 2
_NS = 16
_NW = _NC * _NS
_EPW = _SLICE // _NW
_C = 80
_NCHUNK = _EPW // _C

_R = 6400
_BPS = _SLICE // _R


def _ln_relu(x, g, b):
    m = jnp.mean(x, axis=-1, keepdims=True)
    v = jnp.mean((x - m) ** 2, axis=-1, keepdims=True)
    y = (x - m) * jax.lax.rsqrt(v + 1e-5) * g + b
    return jnp.maximum(y, 0.0)


def _prep_body(sp_ref, vw_ref, gl_ref, g_sp, b_sp, g_v, b_v, g_g, b_g,
               w_sp, w_view, w_glob, b_proj, sp4_ref, vw4_ref, bias4_ref):
    sp = _ln_relu(sp_ref[...], g_sp[...], b_sp[...])
    vw = _ln_relu(vw_ref[...], g_v[...], b_v[...])
    gl = _ln_relu(gl_ref[...], g_g[...], b_g[...])
    dn = (((1,), (1,)), ((), ()))
    sp4_ref[...] = 0.25 * lax.dot_general(sp, w_sp[...], dn,
                                          preferred_element_type=jnp.float32)
    vw4_ref[...] = 0.25 * lax.dot_general(vw, w_view[...], dn,
                                          preferred_element_type=jnp.float32)
    glp = lax.dot_general(gl, w_glob[...], dn,
                          preferred_element_type=jnp.float32)
    bias4_ref[...] = 0.25 * (glp + b_proj[...])


def _sc_gather(s, sp4_hbm, pi_hbm, gsp_hbm, tab, pi_v, buf0, buf1, sem0,
               sem1):
    sid = lax.axis_index("s")
    wid = sid * _NC + lax.axis_index("c")
    obase = wid * _EPW
    ibase = s * _SLICE + obase
    pltpu.sync_copy(pi_hbm.at[pl.ds(ibase, _EPW)], pi_v)

    @pl.when(sid == 0)
    def _load_table():
        pltpu.sync_copy(sp4_hbm, tab)

    plsc.subcore_barrier()

    def start(c, buf, sem):
        pltpu.async_copy(tab.at[pi_v.at[pl.ds(c * _C, _C)]], buf, sem)

    def wait(buf, sem):
        pltpu.make_async_copy(sp4_hbm.at[pl.ds(0, _C)], buf, sem).wait()

    def scatter(c, buf):
        pltpu.sync_copy(buf, gsp_hbm.at[pl.ds(obase + c * _C, _C)])

    start(0, buf0, sem0)

    def body(k, carry):
        c0 = 2 * k
        start(c0 + 1, buf1, sem1)
        wait(buf0, sem0)
        scatter(c0, buf0)
        start(c0 + 2, buf0, sem0)
        wait(buf1, sem1)
        scatter(c0 + 1, buf1)
        return carry

    lax.fori_loop(0, (_NCHUNK - 1) // 2, body, 0)
    wait(buf0, sem0)
    scatter(_NCHUNK - 1, buf0)


def _main_body(x_ref, gsp_ref, vi_ref, w_ref, vwt_ref, bias_ref, *rest):
    o_ref = rest[-1]
    dn = (((1,), (1,)), ((), ()))
    acc = lax.dot_general(x_ref[...], w_ref[...], dn,
                          preferred_element_type=jnp.float32)
    vi = vi_ref[0, 0, :]
    col = lax.broadcasted_iota(jnp.int32, (_R, NVP), 1)
    oh = (col == vi.reshape(_R, 1)).astype(jnp.bfloat16)
    vwterm = lax.dot_general(oh, vwt_ref[...], (((1,), (0,)), ((), ())),
                             preferred_element_type=jnp.float32)
    o_ref[...] = acc * 0.25 + (vwterm + (gsp_ref[...] + bias_ref[...]))


def kernel(proj_values, view_idx, point_idx, scenepoint_features,
           view_features, global_features, ln_sp_g, ln_sp_b, ln_v_g, ln_v_b,
           ln_g_g, ln_g_b, W_proj, b_proj, W_sp, W_view, W_glob):
    row = lambda x: x.reshape(1, D)
    sp4, vw4, bias4 = pl.pallas_call(
        _prep_body,
        out_shape=[
            jax.ShapeDtypeStruct((NP, D), jnp.float32),
            jax.ShapeDtypeStruct((NV, D), jnp.float32),
            jax.ShapeDtypeStruct((1, D), jnp.float32),
        ],
    )(scenepoint_features, view_features, row(global_features),
      row(ln_sp_g), row(ln_sp_b), row(ln_v_g), row(ln_v_b),
      row(ln_g_g), row(ln_g_b), W_sp, W_view, W_glob, row(b_proj))

    mesh = plsc.VectorSubcoreMesh(core_axis_name="c", subcore_axis_name="s")
    pi32 = point_idx.astype(jnp.int32)
    gsp_slices = []
    for s in range(_S):
        gsp_slices.append(pl.kernel(
            functools.partial(_sc_gather, s),
            mesh=mesh,
            out_type=jax.ShapeDtypeStruct((_SLICE, D), jnp.float32),
            scratch_types=[
                pltpu.VMEM_SHARED((NP, D), jnp.float32),
                pltpu.VMEM((_EPW,), jnp.int32),
                pltpu.VMEM((_C, D), jnp.float32),
                pltpu.VMEM((_C, D), jnp.float32),
                pltpu.SemaphoreType.DMA,
                pltpu.SemaphoreType.DMA,
            ],
        )(sp4, pi32))

    vwt = jnp.pad(vw4, ((0, NVP - NV), (0, 0))).astype(jnp.bfloat16)
    vi3 = view_idx.astype(jnp.int32).reshape(NNZ // _R, 1, _R)

    out = None
    for s in range(_S):
        base_specs = [
            pl.BlockSpec((_R, D), lambda i, s=s: (i + s * _BPS, 0)),
            pl.BlockSpec((_R, D), lambda i: (i, 0)),
            pl.BlockSpec((1, 1, _R), lambda i, s=s: (i + s * _BPS, 0, 0)),
            pl.BlockSpec((D, D), lambda i: (0, 0)),
            pl.BlockSpec((NVP, D), lambda i: (0, 0)),
            pl.BlockSpec((1, D), lambda i: (0, 0)),
        ]
        if s == 0:
            main_call = pl.pallas_call(
                _main_body,
                grid=(_BPS,),
                in_specs=base_specs,
                out_specs=pl.BlockSpec((_R, D), lambda i: (i, 0)),
                out_shape=jax.ShapeDtypeStruct((NNZ, D), jnp.float32),
            )
            out = main_call(proj_values, gsp_slices[s], vi3, W_proj, vwt,
                            bias4)
        else:
            main_call = pl.pallas_call(
                _main_body,
                grid=(_BPS,),
                in_specs=base_specs + [pl.BlockSpec(memory_space=pl.ANY)],
                out_specs=pl.BlockSpec(
                    (_R, D), lambda i, s=s: (i + s * _BPS, 0)),
                out_shape=jax.ShapeDtypeStruct((NNZ, D), jnp.float32),
                input_output_aliases={6: 0},
            )
            out = main_call(proj_values, gsp_slices[s], vi3, W_proj, vwt,
                            bias4, out)
    return out


# speedup vs baseline: 6.1899x; 1.0117x over previous
import functools

import jax
import jax.numpy as jnp
from jax import lax
from jax.experimental import pallas as pl
from jax.experimental.pallas import tpu as pltpu
from jax.experimental.pallas import tpu_sc as plsc

NNZ = 320000
NP = 10000
NV = 200
NVP = 256
D = 128

_S = 5
_SLICE = NNZ // _S

_NC = 2
_NS = 16
_NW = _NC * _NS
_EPW = _SLICE // _NW
_C = 80
_NCHUNK = _EPW // _C

_R = 8000
_BPS = _SLICE // _R


def _ln_relu(x, g, b):
    m = jnp.mean(x, axis=-1, keepdims=True)
    v = jnp.mean((x - m) ** 2, axis=-1, keepdims=True)
    y = (x - m) * jax.lax.rsqrt(v + 1e-5) * g + b
    return jnp.maximum(y, 0.0)


def _prep_body(sp_ref, vw_ref, gl_ref, g_sp, b_sp, g_v, b_v, g_g, b_g,
               w_sp, w_view, w_glob, b_proj, sp4_ref, vw4_ref, bias4_ref):
    sp = _ln_relu(sp_ref[...], g_sp[...], b_sp[...])
    vw = _ln_relu(vw_ref[...], g_v[...], b_v[...])
    gl = _ln_relu(gl_ref[...], g_g[...], b_g[...])
    dn = (((1,), (1,)), ((), ()))
    sp4_ref[...] = 0.25 * lax.dot_general(sp, w_sp[...], dn,
                                          preferred_element_type=jnp.float32)
    vw4_ref[...] = 0.25 * lax.dot_general(vw, w_view[...], dn,
                                          preferred_element_type=jnp.float32)
    glp = lax.dot_general(gl, w_glob[...], dn,
                          preferred_element_type=jnp.float32)
    bias4_ref[...] = 0.25 * (glp + b_proj[...])


def _sc_gather(s, sp4_hbm, pi_hbm, gsp_hbm, tab, pi_v, buf0, buf1, sem0,
               sem1):
    sid = lax.axis_index("s")
    wid = sid * _NC + lax.axis_index("c")
    obase = wid * _EPW
    ibase = s * _SLICE + obase
    pltpu.sync_copy(pi_hbm.at[pl.ds(ibase, _EPW)], pi_v)

    @pl.when(sid == 0)
    def _load_table():
        pltpu.sync_copy(sp4_hbm, tab)

    plsc.subcore_barrier()

    def start(c, buf, sem):
        pltpu.async_copy(tab.at[pi_v.at[pl.ds(c * _C, _C)]], buf, sem)

    def wait(buf, sem):
        pltpu.make_async_copy(sp4_hbm.at[pl.ds(0, _C)], buf, sem).wait()

    def scatter(c, buf):
        pltpu.sync_copy(buf, gsp_hbm.at[pl.ds(obase + c * _C, _C)])

    start(0, buf0, sem0)

    def body(k, carry):
        c0 = 2 * k
        start(c0 + 1, buf1, sem1)
        wait(buf0, sem0)
        scatter(c0, buf0)
        start(c0 + 2, buf0, sem0)
        wait(buf1, sem1)
        scatter(c0 + 1, buf1)
        return carry

    lax.fori_loop(0, (_NCHUNK - 1) // 2, body, 0)
    wait(buf0, sem0)
    scatter(_NCHUNK - 1, buf0)


def _main_body(x_ref, gsp_ref, vi_ref, w_ref, vwt_ref, bias_ref, *rest):
    o_ref = rest[-1]
    dn = (((1,), (1,)), ((), ()))
    acc = lax.dot_general(x_ref[...], w_ref[...], dn,
                          preferred_element_type=jnp.float32)
    vi = vi_ref[0, 0, :]
    col = lax.broadcasted_iota(jnp.int32, (_R, NVP), 1)
    oh = (col == vi.reshape(_R, 1)).astype(jnp.bfloat16)
    vwterm = lax.dot_general(oh, vwt_ref[...], (((1,), (0,)), ((), ())),
                             preferred_element_type=jnp.float32)
    o_ref[...] = acc * 0.25 + (vwterm + (gsp_ref[...] + bias_ref[...]))


def kernel(proj_values, view_idx, point_idx, scenepoint_features,
           view_features, global_features, ln_sp_g, ln_sp_b, ln_v_g, ln_v_b,
           ln_g_g, ln_g_b, W_proj, b_proj, W_sp, W_view, W_glob):
    row = lambda x: x.reshape(1, D)
    sp4, vw4, bias4 = pl.pallas_call(
        _prep_body,
        out_shape=[
            jax.ShapeDtypeStruct((NP, D), jnp.float32),
            jax.ShapeDtypeStruct((NV, D), jnp.float32),
            jax.ShapeDtypeStruct((1, D), jnp.float32),
        ],
    )(scenepoint_features, view_features, row(global_features),
      row(ln_sp_g), row(ln_sp_b), row(ln_v_g), row(ln_v_b),
      row(ln_g_g), row(ln_g_b), W_sp, W_view, W_glob, row(b_proj))

    mesh = plsc.VectorSubcoreMesh(core_axis_name="c", subcore_axis_name="s")
    pi32 = point_idx.astype(jnp.int32)
    gsp_slices = []
    for s in range(_S):
        gsp_slices.append(pl.kernel(
            functools.partial(_sc_gather, s),
            mesh=mesh,
            out_type=jax.ShapeDtypeStruct((_SLICE, D), jnp.float32),
            scratch_types=[
                pltpu.VMEM_SHARED((NP, D), jnp.float32),
                pltpu.VMEM((_EPW,), jnp.int32),
                pltpu.VMEM((_C, D), jnp.float32),
                pltpu.VMEM((_C, D), jnp.float32),
                pltpu.SemaphoreType.DMA,
                pltpu.SemaphoreType.DMA,
            ],
        )(sp4, pi32))

    vwt = jnp.pad(vw4, ((0, NVP - NV), (0, 0))).astype(jnp.bfloat16)
    vi3 = view_idx.astype(jnp.int32).reshape(NNZ // _R, 1, _R)

    out = None
    for s in range(_S):
        base_specs = [
            pl.BlockSpec((_R, D), lambda i, s=s: (i + s * _BPS, 0)),
            pl.BlockSpec((_R, D), lambda i: (i, 0)),
            pl.BlockSpec((1, 1, _R), lambda i, s=s: (i + s * _BPS, 0, 0)),
            pl.BlockSpec((D, D), lambda i: (0, 0)),
            pl.BlockSpec((NVP, D), lambda i: (0, 0)),
            pl.BlockSpec((1, D), lambda i: (0, 0)),
        ]
        if s == 0:
            main_call = pl.pallas_call(
                _main_body,
                grid=(_BPS,),
                in_specs=base_specs,
                out_specs=pl.BlockSpec((_R, D), lambda i: (i, 0)),
                out_shape=jax.ShapeDtypeStruct((NNZ, D), jnp.float32),
            )
            out = main_call(proj_values, gsp_slices[s], vi3, W_proj, vwt,
                            bias4)
        else:
            main_call = pl.pallas_call(
                _main_body,
                grid=(_BPS,),
                in_specs=base_specs + [pl.BlockSpec(memory_space=pl.ANY)],
                out_specs=pl.BlockSpec(
                    (_R, D), lambda i, s=s: (i + s * _BPS, 0)),
                out_shape=jax.ShapeDtypeStruct((NNZ, D), jnp.float32),
                input_output_aliases={6: 0},
            )
            out = main_call(proj_values, gsp_slices[s], vi3, W_proj, vwt,
                            bias4, out)
    return out


# speedup vs baseline: 7.0307x; 1.1358x over previous
import functools

import jax
import jax.numpy as jnp
from jax import lax
from jax.experimental import pallas as pl
from jax.experimental.pallas import tpu as pltpu
from jax.experimental.pallas import tpu_sc as plsc

NNZ = 320000
NP = 10000
NV = 200
NVP = 256
D = 128

_S = 5
_SLICE = NNZ // _S

_NC = 2
_NS = 16
_NW = _NC * _NS
_EPW = _SLICE // _NW
_HALF = _EPW // 2
_CP = 40
_NCHUNK = _HALF // _CP

_R = 8000
_RH = _R // 2
_BPS = _SLICE // _R
_WPB = _R // _EPW


def _ln_relu(x, g, b):
    m = jnp.mean(x, axis=-1, keepdims=True)
    v = jnp.mean((x - m) ** 2, axis=-1, keepdims=True)
    y = (x - m) * jax.lax.rsqrt(v + 1e-5) * g + b
    return jnp.maximum(y, 0.0)


def _prep_body(sp_ref, vw_ref, gl_ref, g_sp, b_sp, g_v, b_v, g_g, b_g,
               w_sp, w_view, w_glob, b_proj, sp4_ref, vw4_ref, bias4_ref):
    sp = _ln_relu(sp_ref[...], g_sp[...], b_sp[...])
    vw = _ln_relu(vw_ref[...], g_v[...], b_v[...])
    gl = _ln_relu(gl_ref[...], g_g[...], b_g[...])
    dn = (((1,), (1,)), ((), ()))
    sp4_ref[...] = 0.25 * lax.dot_general(sp, w_sp[...], dn,
                                          preferred_element_type=jnp.float32)
    vw4_ref[...] = 0.25 * lax.dot_general(vw, w_view[...], dn,
                                          preferred_element_type=jnp.float32)
    glp = lax.dot_general(gl, w_glob[...], dn,
                          preferred_element_type=jnp.float32)
    bias4_ref[...] = 0.25 * (glp + b_proj[...])


def _sc_gather(s, sp4_hbm, pi_hbm, gpk_hbm, tab, pi_v, a0, b0, a1, b1,
               pk0, pk1, sem0, sem1):
    sid = lax.axis_index("s")
    wid = sid * _NC + lax.axis_index("c")
    ebase = s * _SLICE + wid * _EPW
    obase = wid * _HALF
    pltpu.sync_copy(pi_hbm.at[pl.ds(ebase, _EPW)], pi_v)

    @pl.when(sid == 0)
    def _load_table():
        pltpu.sync_copy(sp4_hbm, tab)

    plsc.subcore_barrier()

    def start(c, bufa, bufb, sem):
        pltpu.async_copy(tab.at[pi_v.at[pl.ds(c * _CP, _CP)]], bufa, sem)
        pltpu.async_copy(tab.at[pi_v.at[pl.ds(_HALF + c * _CP, _CP)]],
                         bufb, sem)

    def wait2(bufa, bufb, sem):
        pltpu.make_async_copy(sp4_hbm.at[pl.ds(0, _CP)], bufa, sem).wait()
        pltpu.make_async_copy(sp4_hbm.at[pl.ds(0, _CP)], bufb, sem).wait()

    def pack(bufa, bufb, pk):
        def prow(r, carry):
            for v in range(8):
                a = bufa[r, pl.ds(16 * v, 16)]
                b = bufb[r, pl.ds(16 * v, 16)]
                w = plsc.bitcast(
                    plsc.pack(a, b, format=plsc.PackFormat.INTERLEAVED),
                    jnp.int32)
                pk[r, pl.ds(16 * v, 16)] = w
            return carry
        lax.fori_loop(0, _CP, prow, 0)

    def scatter(c, pk):
        pltpu.sync_copy(pk, gpk_hbm.at[pl.ds(obase + c * _CP, _CP)])

    start(0, a0, b0, sem0)

    def body(k, carry):
        c0 = 2 * k
        start(c0 + 1, a1, b1, sem1)
        wait2(a0, b0, sem0)
        pack(a0, b0, pk0)
        scatter(c0, pk0)
        start(c0 + 2, a0, b0, sem0)
        wait2(a1, b1, sem1)
        pack(a1, b1, pk1)
        scatter(c0 + 1, pk1)
        return carry

    lax.fori_loop(0, (_NCHUNK - 1) // 2, body, 0)
    wait2(a0, b0, sem0)
    pack(a0, b0, pk0)
    scatter(_NCHUNK - 1, pk0)


def _main_body(x_ref, gpk_ref, vi_ref, w_ref, vwt_ref, bias_ref, *rest):
    o_ref = rest[-1]
    dn = (((1,), (1,)), ((), ()))
    acc = lax.dot_general(x_ref[...], w_ref[...], dn,
                          preferred_element_type=jnp.float32)
    vi = vi_ref[0, 0, :]
    col = lax.broadcasted_iota(jnp.int32, (_R, NVP), 1)
    oh = (col == vi.reshape(_R, 1)).astype(jnp.bfloat16)
    vwterm = lax.dot_general(oh, vwt_ref[...], (((1,), (0,)), ((), ())),
                             preferred_element_type=jnp.float32)
    comb = acc * 0.25 + (vwterm + bias_ref[...])
    gp = gpk_ref[...]
    af = lax.bitcast_convert_type(gp << 16, jnp.float32)
    bf = lax.bitcast_convert_type((gp >> 16) << 16, jnp.float32)
    for j in range(_WPB):
        lo = _EPW * j
        ph = _HALF * j
        o_ref[pl.ds(lo, _HALF), :] = (
            comb[lo:lo + _HALF, :] + af[ph:ph + _HALF, :])
        o_ref[pl.ds(lo + _HALF, _HALF), :] = (
            comb[lo + _HALF:lo + _EPW, :] + bf[ph:ph + _HALF, :])


def kernel(proj_values, view_idx, point_idx, scenepoint_features,
           view_features, global_features, ln_sp_g, ln_sp_b, ln_v_g, ln_v_b,
           ln_g_g, ln_g_b, W_proj, b_proj, W_sp, W_view, W_glob):
    row = lambda x: x.reshape(1, D)
    sp4, vw4, bias4 = pl.pallas_call(
        _prep_body,
        out_shape=[
            jax.ShapeDtypeStruct((NP, D), jnp.float32),
            jax.ShapeDtypeStruct((NV, D), jnp.float32),
            jax.ShapeDtypeStruct((1, D), jnp.float32),
        ],
    )(scenepoint_features, view_features, row(global_features),
      row(ln_sp_g), row(ln_sp_b), row(ln_v_g), row(ln_v_b),
      row(ln_g_g), row(ln_g_b), W_sp, W_view, W_glob, row(b_proj))

    mesh = plsc.VectorSubcoreMesh(core_axis_name="c", subcore_axis_name="s")
    pi32 = point_idx.astype(jnp.int32)
    gpk_slices = []
    for s in range(_S):
        gpk_slices.append(pl.kernel(
            functools.partial(_sc_gather, s),
            mesh=mesh,
            compiler_params=pltpu.CompilerParams(needs_layout_passes=False),
            out_type=jax.ShapeDtypeStruct((_SLICE // 2, D), jnp.int32),
            scratch_types=[
                pltpu.VMEM_SHARED((NP, D), jnp.float32),
                pltpu.VMEM((_EPW,), jnp.int32),
                pltpu.VMEM((_CP, D), jnp.float32),
                pltpu.VMEM((_CP, D), jnp.float32),
                pltpu.VMEM((_CP, D), jnp.float32),
                pltpu.VMEM((_CP, D), jnp.float32),
                pltpu.VMEM((_CP, D), jnp.int32),
                pltpu.VMEM((_CP, D), jnp.int32),
                pltpu.SemaphoreType.DMA,
                pltpu.SemaphoreType.DMA,
            ],
        )(sp4, pi32))

    vwt = jnp.pad(vw4, ((0, NVP - NV), (0, 0))).astype(jnp.bfloat16)
    vi3 = view_idx.astype(jnp.int32).reshape(NNZ // _R, 1, _R)

    out = None
    for s in range(_S):
        base_specs = [
            pl.BlockSpec((_R, D), lambda i, s=s: (i + s * _BPS, 0)),
            pl.BlockSpec((_RH, D), lambda i: (i, 0)),
            pl.BlockSpec((1, 1, _R), lambda i, s=s: (i + s * _BPS, 0, 0)),
            pl.BlockSpec((D, D), lambda i: (0, 0)),
            pl.BlockSpec((NVP, D), lambda i: (0, 0)),
            pl.BlockSpec((1, D), lambda i: (0, 0)),
        ]
        if s == 0:
            main_call = pl.pallas_call(
                _main_body,
                grid=(_BPS,),
                in_specs=base_specs,
                out_specs=pl.BlockSpec((_R, D), lambda i: (i, 0)),
                out_shape=jax.ShapeDtypeStruct((NNZ, D), jnp.float32),
            )
            out = main_call(proj_values, gpk_slices[s], vi3, W_proj, vwt,
                            bias4)
        else:
            main_call = pl.pallas_call(
                _main_body,
                grid=(_BPS,),
                in_specs=base_specs + [pl.BlockSpec(memory_space=pl.ANY)],
                out_specs=pl.BlockSpec(
                    (_R, D), lambda i, s=s: (i + s * _BPS, 0)),
                out_shape=jax.ShapeDtypeStruct((NNZ, D), jnp.float32),
                input_output_aliases={6: 0},
            )
            out = main_call(proj_values, gpk_slices[s], vi3, W_proj, vwt,
                            bias4, out)
    return out


# speedup vs baseline: 7.2245x; 1.0276x over previous
import functools

import jax
import jax.numpy as jnp
from jax import lax
from jax.experimental import pallas as pl
from jax.experimental.pallas import tpu as pltpu
from jax.experimental.pallas import tpu_sc as plsc

NNZ = 320000
NP = 10000
NV = 200
NVP = 256
D = 128

_S = 5
_SLICE = NNZ // _S

_NC = 2
_NS = 16
_NW = _NC * _NS
_EPW = _SLICE // _NW
_HALF = _EPW // 2
_CP = 40
_NCHUNK = _HALF // _CP

_R = 16000
_RH = _R // 2
_BPS = _SLICE // _R
_WPB = _R // _EPW


def _ln_relu(x, g, b):
    m = jnp.mean(x, axis=-1, keepdims=True)
    v = jnp.mean((x - m) ** 2, axis=-1, keepdims=True)
    y = (x - m) * jax.lax.rsqrt(v + 1e-5) * g + b
    return jnp.maximum(y, 0.0)


def _prep_body(sp_ref, vw_ref, gl_ref, g_sp, b_sp, g_v, b_v, g_g, b_g,
               w_sp, w_view, w_glob, b_proj, sp4_ref, vw4_ref, bias4_ref):
    sp = _ln_relu(sp_ref[...], g_sp[...], b_sp[...])
    vw = _ln_relu(vw_ref[...], g_v[...], b_v[...])
    gl = _ln_relu(gl_ref[...], g_g[...], b_g[...])
    dn = (((1,), (1,)), ((), ()))
    sp4_ref[...] = 0.25 * lax.dot_general(sp, w_sp[...], dn,
                                          preferred_element_type=jnp.float32)
    vw4_ref[...] = 0.25 * lax.dot_general(vw, w_view[...], dn,
                                          preferred_element_type=jnp.float32)
    glp = lax.dot_general(gl, w_glob[...], dn,
                          preferred_element_type=jnp.float32)
    bias4_ref[...] = 0.25 * (glp + b_proj[...])


def _sc_gather(s, sp4_hbm, pi_hbm, gpk_hbm, tab, pi_v, a0, b0, a1, b1,
               pk0, pk1, sem0, sem1):
    sid = lax.axis_index("s")
    wid = sid * _NC + lax.axis_index("c")
    ebase = s * _SLICE + wid * _EPW
    obase = wid * _HALF
    pltpu.sync_copy(pi_hbm.at[pl.ds(ebase, _EPW)], pi_v)

    @pl.when(sid == 0)
    def _load_table():
        pltpu.sync_copy(sp4_hbm, tab)

    plsc.subcore_barrier()

    def start(c, bufa, bufb, sem):
        pltpu.async_copy(tab.at[pi_v.at[pl.ds(c * _CP, _CP)]], bufa, sem)
        pltpu.async_copy(tab.at[pi_v.at[pl.ds(_HALF + c * _CP, _CP)]],
                         bufb, sem)

    def wait2(bufa, bufb, sem):
        pltpu.make_async_copy(sp4_hbm.at[pl.ds(0, _CP)], bufa, sem).wait()
        pltpu.make_async_copy(sp4_hbm.at[pl.ds(0, _CP)], bufb, sem).wait()

    def pack(bufa, bufb, pk):
        def prow(r, carry):
            for v in range(8):
                a = bufa[r, pl.ds(16 * v, 16)]
                b = bufb[r, pl.ds(16 * v, 16)]
                w = plsc.bitcast(
                    plsc.pack(a, b, format=plsc.PackFormat.INTERLEAVED),
                    jnp.int32)
                pk[r, pl.ds(16 * v, 16)] = w
            return carry
        lax.fori_loop(0, _CP, prow, 0)

    def scatter(c, pk):
        pltpu.sync_copy(pk, gpk_hbm.at[pl.ds(obase + c * _CP, _CP)])

    start(0, a0, b0, sem0)

    def body(k, carry):
        c0 = 2 * k
        start(c0 + 1, a1, b1, sem1)
        wait2(a0, b0, sem0)
        pack(a0, b0, pk0)
        scatter(c0, pk0)
        start(c0 + 2, a0, b0, sem0)
        wait2(a1, b1, sem1)
        pack(a1, b1, pk1)
        scatter(c0 + 1, pk1)
        return carry

    lax.fori_loop(0, (_NCHUNK - 1) // 2, body, 0)
    wait2(a0, b0, sem0)
    pack(a0, b0, pk0)
    scatter(_NCHUNK - 1, pk0)


def _main_body(x_ref, gpk_ref, vi_ref, w_ref, vwt_ref, bias_ref, *rest):
    o_ref = rest[-1]
    dn = (((1,), (1,)), ((), ()))
    acc = lax.dot_general(x_ref[...], w_ref[...], dn,
                          preferred_element_type=jnp.float32)
    vi = vi_ref[0, 0, :]
    col = lax.broadcasted_iota(jnp.int32, (_R, NVP), 1)
    oh = (col == vi.reshape(_R, 1)).astype(jnp.bfloat16)
    vwterm = lax.dot_general(oh, vwt_ref[...], (((1,), (0,)), ((), ())),
                             preferred_element_type=jnp.float32)
    comb = acc * 0.25 + (vwterm + bias_ref[...])
    gp = gpk_ref[...]
    af = lax.bitcast_convert_type(gp << 16, jnp.float32)
    bf = lax.bitcast_convert_type((gp >> 16) << 16, jnp.float32)
    for j in range(_WPB):
        lo = _EPW * j
        ph = _HALF * j
        o_ref[pl.ds(lo, _HALF), :] = (
            comb[lo:lo + _HALF, :] + af[ph:ph + _HALF, :])
        o_ref[pl.ds(lo + _HALF, _HALF), :] = (
            comb[lo + _HALF:lo + _EPW, :] + bf[ph:ph + _HALF, :])


def kernel(proj_values, view_idx, point_idx, scenepoint_features,
           view_features, global_features, ln_sp_g, ln_sp_b, ln_v_g, ln_v_b,
           ln_g_g, ln_g_b, W_proj, b_proj, W_sp, W_view, W_glob):
    row = lambda x: x.reshape(1, D)
    sp4, vw4, bias4 = pl.pallas_call(
        _prep_body,
        out_shape=[
            jax.ShapeDtypeStruct((NP, D), jnp.float32),
            jax.ShapeDtypeStruct((NV, D), jnp.float32),
            jax.ShapeDtypeStruct((1, D), jnp.float32),
        ],
    )(scenepoint_features, view_features, row(global_features),
      row(ln_sp_g), row(ln_sp_b), row(ln_v_g), row(ln_v_b),
      row(ln_g_g), row(ln_g_b), W_sp, W_view, W_glob, row(b_proj))

    mesh = plsc.VectorSubcoreMesh(core_axis_name="c", subcore_axis_name="s")
    pi32 = point_idx.astype(jnp.int32)
    gpk_slices = []
    for s in range(_S):
        gpk_slices.append(pl.kernel(
            functools.partial(_sc_gather, s),
            mesh=mesh,
            compiler_params=pltpu.CompilerParams(needs_layout_passes=False),
            out_type=jax.ShapeDtypeStruct((_SLICE // 2, D), jnp.int32),
            scratch_types=[
                pltpu.VMEM_SHARED((NP, D), jnp.float32),
                pltpu.VMEM((_EPW,), jnp.int32),
                pltpu.VMEM((_CP, D), jnp.float32),
                pltpu.VMEM((_CP, D), jnp.float32),
                pltpu.VMEM((_CP, D), jnp.float32),
                pltpu.VMEM((_CP, D), jnp.float32),
                pltpu.VMEM((_CP, D), jnp.int32),
                pltpu.VMEM((_CP, D), jnp.int32),
                pltpu.SemaphoreType.DMA,
                pltpu.SemaphoreType.DMA,
            ],
        )(sp4, pi32))

    vwt = jnp.pad(vw4, ((0, NVP - NV), (0, 0))).astype(jnp.bfloat16)
    vi3 = view_idx.astype(jnp.int32).reshape(NNZ // _R, 1, _R)

    out = None
    for s in range(_S):
        base_specs = [
            pl.BlockSpec((_R, D), lambda i, s=s: (i + s * _BPS, 0)),
            pl.BlockSpec((_RH, D), lambda i: (i, 0)),
            pl.BlockSpec((1, 1, _R), lambda i, s=s: (i + s * _BPS, 0, 0)),
            pl.BlockSpec((D, D), lambda i: (0, 0)),
            pl.BlockSpec((NVP, D), lambda i: (0, 0)),
            pl.BlockSpec((1, D), lambda i: (0, 0)),
        ]
        if s == 0:
            main_call = pl.pallas_call(
                _main_body,
                grid=(_BPS,),
                in_specs=base_specs,
                out_specs=pl.BlockSpec((_R, D), lambda i: (i, 0)),
                out_shape=jax.ShapeDtypeStruct((NNZ, D), jnp.float32),
            )
            out = main_call(proj_values, gpk_slices[s], vi3, W_proj, vwt,
                            bias4)
        else:
            main_call = pl.pallas_call(
                _main_body,
                grid=(_BPS,),
                in_specs=base_specs + [pl.BlockSpec(memory_space=pl.ANY)],
                out_specs=pl.BlockSpec(
                    (_R, D), lambda i, s=s: (i + s * _BPS, 0)),
                out_shape=jax.ShapeDtypeStruct((NNZ, D), jnp.float32),
                input_output_aliases={6: 0},
            )
            out = main_call(proj_values, gpk_slices[s], vi3, W_proj, vwt,
                            bias4, out)
    return out
